# baseline (speedup 1.0000x reference)
import functools

import jax
import jax.numpy as jnp
from jax import lax
from jax.experimental import pallas as pl
from jax.experimental.pallas import tpu as pltpu
from jax.experimental.pallas import tpu_sc as plsc

D_MODEL = 64
SCALE = 8.0

NC = 2
NS = 16
NW = NC * NS
LANES = 16

SB = 8
BB = 128


@jax.jit
def _embed_lookup(x4, table):
    n_sblk, n_bblk = x4.shape[0], x4.shape[1]
    n_items = n_sblk * n_bblk
    per_w = n_items // NW
    mesh = plsc.VectorSubcoreMesh(
        core_axis_name="c", subcore_axis_name="s", num_cores=NC, num_subcores=NS
    )

    @functools.partial(
        pl.kernel,
        out_type=jax.ShapeDtypeStruct(
            (n_sblk * SB, D_MODEL // 8, n_bblk, 8, BB), jnp.float32
        ),
        mesh=mesh,
        scratch_types=[
            pltpu.VMEM((SB, BB), jnp.int32),
            pltpu.VMEM((2, BB, D_MODEL), jnp.float32),
            pltpu.VMEM((2, D_MODEL // 8, 8, BB), jnp.float32),
            pltpu.SemaphoreType.DMA((2,)),
            pltpu.SemaphoreType.DMA((2,)),
        ],
        compiler_params=pltpu.CompilerParams(
            use_tc_tiling_on_sc=False, needs_layout_passes=False
        ),
    )
    def body(x_hbm, table_hbm, out_hbm, idx_v, rows_v, obuf, sem_g, sem_o):
        wid = lax.axis_index("s") * NC + lax.axis_index("c")
        iota = lax.iota(jnp.int32, LANES)

        def do_item(i, _):
            it = wid * per_w + i
            st = it // n_bblk
            bt = it % n_bblk
            pltpu.sync_copy(x_hbm.at[st, bt], idx_v)

            def start_gather(k):
                return pltpu.async_copy(
                    table_hbm.at[idx_v.at[k]], rows_v.at[k % 2], sem_g.at[k % 2]
                )

            def start_out(k):
                s = st * SB + k
                return pltpu.async_copy(
                    obuf.at[k % 2], out_hbm.at[s, :, bt], sem_o.at[k % 2]
                )

            gathers = [start_gather(0)]
            outs = []
            for k in range(SB):
                if k + 1 < SB:
                    gathers.append(start_gather(k + 1))
                gathers[k].wait()
                if k >= 2:
                    outs[k - 2].wait()
                j = k % 2

                def tpose(d, _):
                    dvec = jnp.full((LANES,), d, jnp.int32)
                    for g in range(BB // LANES):
                        v = plsc.load_gather(
                            rows_v.at[j], [iota + (g * LANES), dvec]
                        )
                        obuf[j, d // 8, d % 8, pl.ds(g * LANES, LANES)] = (
                            v * SCALE
                        )
                    return ()

                lax.fori_loop(0, D_MODEL, tpose, ())
                outs.append(start_out(k))
            outs[SB - 2].wait()
            outs[SB - 1].wait()
            return ()

        lax.fori_loop(0, per_w, do_item, ())

    return body(x4, table)


def kernel(x, table):
    b, s = x.shape
    n_sblk, n_bblk = s // SB, b // BB
    x4 = (
        x.T.astype(jnp.int32)
        .reshape(n_sblk, SB, n_bblk, BB)
        .transpose(0, 2, 1, 3)
    )
    o5 = _embed_lookup(x4, table)
    return o5.transpose(2, 4, 0, 1, 3).reshape(b, s, D_MODEL)
